# Initial kernel scaffold; baseline (speedup 1.0000x reference)
import functools

import jax
import jax.numpy as jnp
from jax import lax
from jax.experimental import pallas as pl
from jax.experimental.pallas import tpu as pltpu
from jax.experimental.pallas import tpu_sc as plsc

N, V, H, W = 4, 50000, 512, 512
NV = N * V
F = 4 * 100000
PX = N * H * W

NC, NS, L = 2, 16, 16
NW = NC * NS

VPAD = 200704
RPT = VPAD // NS
RC = 1568
FPAD = 409600
FPT = FPAD // NW
FC = 1280
VIS_DUMMY = FPAD
VIS_SZ = FPAD + 8
PC = 2048
PPT = PX // NS
PB = 2048
PXT = PX // NW

_GRID_SCALE = jnp.float32(2.0 / 511.0)

_mesh = plsc.VectorSubcoreMesh(core_axis_name="c", subcore_axis_name="s")


def _fill(ref, n, val, dtype):
    cst = jnp.full((L,), val, dtype)

    def body(g, _):
        ref[pl.ds(g * L, L)] = cst
        return 0

    lax.fori_loop(0, n // L, body, 0)


@functools.partial(
    pl.kernel,
    out_type=jax.ShapeDtypeStruct((FPAD, 16), jnp.float32),
    mesh=_mesh,
    scratch_types=[
        pltpu.VMEM((RC * 3,), jnp.float32),
        pltpu.VMEM((RC * 3,), jnp.float32),
        pltpu.VMEM((RC, 4), jnp.float32),
        pltpu.VMEM((6400,), jnp.int32),
        pltpu.VMEM((PC,), jnp.int32),
        pltpu.VMEM((PC,), jnp.int32),
        pltpu.VMEM((PC,), jnp.int32),
        pltpu.VMEM((FC, 3), jnp.int32),
        pltpu.VMEM((FC,), jnp.int32),
        pltpu.VMEM((FC, 4), jnp.float32),
        pltpu.VMEM((FC,), jnp.int32),
        pltpu.VMEM((FC, 16), jnp.float32),
        pltpu.VMEM_SHARED((VPAD, 4), jnp.float32),
        pltpu.VMEM_SHARED((VIS_SZ,), jnp.int32),
        pltpu.SemaphoreType.DMA,
    ],
)
def _build_table(vs_hbm, vt_hbm, faces_hbm, p2fs_hbm, tbl_hbm,
                 vsb, vtb, padb, zb, onesb, p2fb, sidxb, facesb, gidxb,
                 grows, visb, tchunk, flow_sh, vis_sh, sem):
    sid = lax.axis_index("s")
    wid = lax.axis_index("c") * NS + sid
    iota = lax.iota(jnp.int32, L)

    _fill(zb, 6400, 0, jnp.int32)
    for i in range(4):
        pltpu.sync_copy(zb, vis_sh.at[pl.ds(sid * 25600 + i * 6400, 6400)])

    def sub_chunk(c, _):
        r0 = sid * RPT + c * RC
        pltpu.sync_copy(vs_hbm.at[pl.ds(r0 * 3, RC * 3)], vsb)
        pltpu.sync_copy(vt_hbm.at[pl.ds(r0 * 3, RC * 3)], vtb)

        def grp(g, _):
            base = g * L
            lv = base + iota
            d = vtb[pl.ds(base, L)] - vsb[pl.ds(base, L)]
            l3 = (lv * 43691) >> 17
            plsc.store_scatter(padb, [l3, lv - l3 * 3], d)
            return 0

        lax.fori_loop(0, RC * 3 // L, grp, 0)
        pltpu.sync_copy(padb, flow_sh.at[pl.ds(r0, RC)])
        return 0

    lax.fori_loop(0, RPT // RC, sub_chunk, 0)

    plsc.subcore_barrier()

    _fill(onesb, PC, 1, jnp.int32)

    def sc_chunk(c, _):
        off = sid * PPT + c * PC
        pltpu.sync_copy(p2fs_hbm.at[pl.ds(off, PC)], p2fb)

        def grp(g, _):
            v = p2fb[pl.ds(g * L, L)]
            sidxb[pl.ds(g * L, L)] = jnp.where(v < 0, VIS_DUMMY, v)
            return 0

        lax.fori_loop(0, PC // L, grp, 0)
        pltpu.sync_copy(onesb, vis_sh.at[sidxb], add=True)
        return 0

    lax.fori_loop(0, PPT // PC, sc_chunk, 0)

    plsc.subcore_barrier()

    def tb_chunk(c, _):
        f0 = wid * FPT + c * FC
        pltpu.sync_copy(faces_hbm.at[pl.ds(f0, FC)], facesb)
        pltpu.sync_copy(vis_sh.at[pl.ds(f0, FC)], visb)
        for j in range(3):
            def gi(g, _):
                base = g * L
                gidxb[pl.ds(base, L)] = plsc.load_gather(
                    facesb, [base + iota, jnp.full((L,), j, jnp.int32)])
                return 0

            lax.fori_loop(0, FC // L, gi, 0)
            pltpu.async_copy(flow_sh.at[gidxb], grows, sem).wait()

            def cp(g, _):
                base = g * L
                for col in range(3):
                    x = plsc.load_gather(
                        grows, [base + iota, jnp.full((L,), col, jnp.int32)])
                    plsc.store_scatter(
                        tchunk,
                        [base + iota, jnp.full((L,), 4 * j + col, jnp.int32)],
                        x)
                return 0

            lax.fori_loop(0, FC // L, cp, 0)

        def vv(g, _):
            base = g * L
            cnt = visb[pl.ds(base, L)]
            vf = jnp.where(cnt > 0, jnp.float32(1.0), jnp.float32(0.0))
            plsc.store_scatter(
                tchunk, [base + iota, jnp.full((L,), 12, jnp.int32)], vf)
            return 0

        lax.fori_loop(0, FC // L, vv, 0)
        pltpu.sync_copy(tchunk, tbl_hbm.at[pl.ds(f0, FC)])
        return 0

    lax.fori_loop(0, FPT // FC, tb_chunk, 0)


@functools.partial(
    pl.kernel,
    out_type=jax.ShapeDtypeStruct((PX, 4), jnp.float32),
    mesh=_mesh,
    scratch_types=[
        pltpu.VMEM((PB,), jnp.int32),
        pltpu.VMEM((PB * 3,), jnp.float32),
        pltpu.VMEM((PB,), jnp.int32),
        pltpu.VMEM((PB, 16), jnp.float32),
        pltpu.VMEM((PB, 4), jnp.float32),
        pltpu.SemaphoreType.DMA,
    ],
)
def _render(tbl_hbm, p2ft_hbm, bary_hbm, out_hbm,
            p2fb, baryb, idxb, rows, outb, sem):
    wid = lax.axis_index("c") * NS + lax.axis_index("s")
    iota = lax.iota(jnp.int32, L)

    def px_chunk(c, _):
        p0 = wid * PXT + c * PB
        pltpu.sync_copy(p2ft_hbm.at[pl.ds(p0, PB)], p2fb)
        pltpu.sync_copy(bary_hbm.at[pl.ds(p0 * 3, PB * 3)], baryb)

        def ig(g, _):
            v = p2fb[pl.ds(g * L, L)]
            idxb[pl.ds(g * L, L)] = jnp.maximum(v, 0)
            return 0

        lax.fori_loop(0, PB // L, ig, 0)
        pltpu.async_copy(tbl_hbm.at[idxb], rows, sem).wait()

        def grp(g, _):
            base = g * L
            r = base + iota

            def colg(cc):
                return plsc.load_gather(
                    rows, [r, jnp.full((L,), cc, jnp.int32)])

            b0 = plsc.load_gather(baryb, [r * 3])
            b1 = plsc.load_gather(baryb, [r * 3 + 1])
            b2 = plsc.load_gather(baryb, [r * 3 + 2])
            ox = b0 * colg(0) + b1 * colg(4) + b2 * colg(8)
            oy = b0 * colg(1) + b1 * colg(5) + b2 * colg(9)
            oz = b0 * colg(2) + b1 * colg(6) + b2 * colg(10)
            ov = colg(12)
            pm = p2fb[pl.ds(base, L)] >= 0
            q = p0 + r
            gx = (q & 511).astype(jnp.float32) * _GRID_SCALE - 1.0
            gy = ((q >> 9) & 511).astype(jnp.float32) * _GRID_SCALE - 1.0
            zero = jnp.zeros((L,), jnp.float32)
            ox = jnp.where(pm, ox, zero) + gx
            oy = jnp.where(pm, oy, zero) + gy
            oz = jnp.where(pm, oz, zero)
            ov = jnp.where(pm, ov, zero)
            for cc, vec in ((0, ox), (1, oy), (2, oz), (3, ov)):
                plsc.store_scatter(
                    outb, [r, jnp.full((L,), cc, jnp.int32)], vec)
            return 0

        lax.fori_loop(0, PB // L, grp, 0)
        pltpu.sync_copy(outb, out_hbm.at[pl.ds(p0, PB)])
        return 0

    lax.fori_loop(0, PXT // PB, px_chunk, 0)


def kernel(verts_source_ndc, verts_target_ndc, faces, pix_to_face_source,
           pix_to_face_target, bary_coords):
    vs = jnp.concatenate(
        [verts_source_ndc.reshape(-1),
         jnp.zeros(((VPAD - NV) * 3,), jnp.float32)])
    vt = jnp.concatenate(
        [verts_target_ndc.reshape(-1),
         jnp.zeros(((VPAD - NV) * 3,), jnp.float32)])
    faces_pad = jnp.concatenate(
        [faces, jnp.zeros((FPAD - F, 3), jnp.int32)], axis=0)
    p2fs = pix_to_face_source.reshape(-1)
    p2ft = pix_to_face_target.reshape(-1)
    bary = bary_coords.reshape(-1)

    tbl = _build_table(vs, vt, faces_pad, p2fs)
    out = _render(tbl, p2ft, bary)
    return out.reshape(N, H, W, 4)


# speedup vs baseline: 9.9174x; 9.9174x over previous
import functools

import jax
import jax.numpy as jnp
from jax import lax
from jax.experimental import pallas as pl
from jax.experimental.pallas import tpu as pltpu
from jax.experimental.pallas import tpu_sc as plsc

N, V, H, W = 4, 50000, 512, 512
NV = N * V
F = 4 * 100000
PX = N * H * W

NC, NS, L = 2, 16, 16
NW = NC * NS

VPAD = 200704
RPT = VPAD // NW
RC = 1568
FPAD = 409600
FPT = FPAD // NW
FC = 1280
VIS_DUMMY = FPAD
PC = 2048
PPT = PX // NS
PB = 2048
PXT = PX // NW

_GRID_SCALE = 2.0 / 511.0

_mesh = plsc.VectorSubcoreMesh(core_axis_name="c", subcore_axis_name="s")
_params = pltpu.CompilerParams(
    needs_layout_passes=False, use_tc_tiling_on_sc=False)


@functools.partial(
    pl.kernel,
    out_type=(
        jax.ShapeDtypeStruct((VPAD * 16,), jnp.float32),
        jax.ShapeDtypeStruct((VPAD * 16,), jnp.float32),
        jax.ShapeDtypeStruct((VPAD * 16,), jnp.float32),
    ),
    mesh=_mesh,
    compiler_params=_params,
    scratch_types=[
        pltpu.VMEM((RC * 3,), jnp.float32),
        pltpu.VMEM((RC * 3,), jnp.float32),
        pltpu.VMEM((RC * 16,), jnp.float32),
        pltpu.VMEM((RC * 16,), jnp.float32),
        pltpu.VMEM((RC * 16,), jnp.float32),
    ],
)
def _flow_tables(vs_hbm, vt_hbm, zeros_hbm, fa_hbm, fb_hbm, fc_hbm,
                 vsb, vtb, padA, padB, padC):
    wid = lax.axis_index("c") * NS + lax.axis_index("s")
    iota = lax.iota(jnp.int32, L)

    pltpu.sync_copy(zeros_hbm, padA)
    pltpu.sync_copy(zeros_hbm, padB)
    pltpu.sync_copy(zeros_hbm, padC)

    def chunk(c, _):
        r0 = wid * RPT + c * RC
        pltpu.sync_copy(vs_hbm.at[pl.ds(r0 * 3, RC * 3)], vsb)
        pltpu.sync_copy(vt_hbm.at[pl.ds(r0 * 3, RC * 3)], vtb)

        def grp(g, _):
            base = g * L
            lv = base + iota
            d = vtb[pl.ds(base, L)] - vsb[pl.ds(base, L)]
            l3 = (lv * 43691) >> 17
            ia = l3 * 13 + lv
            plsc.store_scatter(padA, [ia], d)
            plsc.store_scatter(padB, [ia + 4], d)
            plsc.store_scatter(padC, [ia + 8], d)
            return 0

        lax.fori_loop(0, RC * 3 // L, grp, 0)
        pltpu.sync_copy(padA, fa_hbm.at[pl.ds(r0 * 16, RC * 16)])
        pltpu.sync_copy(padB, fb_hbm.at[pl.ds(r0 * 16, RC * 16)])
        pltpu.sync_copy(padC, fc_hbm.at[pl.ds(r0 * 16, RC * 16)])
        return 0

    lax.fori_loop(0, RPT // RC, chunk, 0)


@functools.partial(
    pl.kernel,
    out_type=jax.ShapeDtypeStruct((FPAD * 16,), jnp.float32),
    mesh=_mesh,
    compiler_params=_params,
    scratch_types=[
        pltpu.VMEM((PC,), jnp.float32),
        pltpu.VMEM((PC,), jnp.int32),
        pltpu.VMEM((PC,), jnp.int32),
        pltpu.VMEM((FC * 3,), jnp.int32),
        pltpu.VMEM((FC,), jnp.int32),
        pltpu.VMEM((FC,), jnp.int32),
        pltpu.VMEM((FC,), jnp.int32),
        pltpu.VMEM((FC, 16), jnp.float32),
        pltpu.VMEM((FC, 16), jnp.float32),
        pltpu.VMEM((FC, 16), jnp.float32),
        pltpu.VMEM((FC,), jnp.float32),
        pltpu.VMEM((FC * 16,), jnp.float32),
        pltpu.VMEM_SHARED((FPAD + 8,), jnp.float32),
        pltpu.SemaphoreType.DMA,
    ],
)
def _build_table(faces_hbm, p2fs_hbm, fa_hbm, fb_hbm, fc_hbm,
                 zeros_hbm, tbl_hbm,
                 ones1v, p2fb, sidxb, facesb, gidx0, gidx1, gidx2,
                 growsA, growsB, growsC, visb, tchunkf, vis1, sem):
    sid = lax.axis_index("s")
    wid = lax.axis_index("c") * NS + sid
    iota = lax.iota(jnp.int32, L)

    zrows = FPAD // NS
    pltpu.sync_copy(zeros_hbm, vis1.at[pl.ds(sid * zrows, zrows)])
    onev = jnp.full((L,), 1.0, jnp.float32)

    def of(g, _):
        ones1v[pl.ds(g * L, L)] = onev
        return 0

    lax.fori_loop(0, PC // L, of, 0)
    plsc.subcore_barrier()

    def sc_chunk(c, _):
        off = sid * PPT + c * PC
        pltpu.sync_copy(p2fs_hbm.at[pl.ds(off, PC)], p2fb)

        def grp(g, _):
            v = p2fb[pl.ds(g * L, L)]
            sidxb[pl.ds(g * L, L)] = jnp.where(v < 0, VIS_DUMMY, v)
            return 0

        lax.fori_loop(0, PC // L, grp, 0)
        pltpu.sync_copy(ones1v, vis1.at[sidxb], add=True)
        return 0

    lax.fori_loop(0, PPT // PC, sc_chunk, 0)
    plsc.subcore_barrier()

    gidx = (gidx0, gidx1, gidx2)

    def tb_chunk(c, _):
        f0 = wid * FPT + c * FC
        pltpu.sync_copy(faces_hbm.at[pl.ds(f0 * 3, FC * 3)], facesb)
        for j in range(3):
            def gi(g, _):
                base = g * L
                v = plsc.load_gather(facesb, [(base + iota) * 3 + j])
                gidx[j][pl.ds(base, L)] = v
                return 0

            lax.fori_loop(0, FC // L, gi, 0)
        cps = [
            pltpu.async_copy(fa_hbm.at[gidx0], growsA, sem),
            pltpu.async_copy(fb_hbm.at[gidx1], growsB, sem),
            pltpu.async_copy(fc_hbm.at[gidx2], growsC, sem),
        ]
        for cp in cps:
            cp.wait()
        pltpu.sync_copy(vis1.at[pl.ds(f0, FC)], visb)

        def face4(k, _):
            for u in range(4):
                p = k * 4 + u
                row = growsA[p, :] + growsB[p, :] + growsC[p, :]
                tchunkf[pl.ds(p * 16, L)] = row
            return 0

        lax.fori_loop(0, FC // 4, face4, 0)

        def vv(g, _):
            base = g * L
            cnt = visb[pl.ds(base, L)]
            plsc.store_scatter(tchunkf, [(base + iota) * 16 + 12], cnt)
            return 0

        lax.fori_loop(0, FC // L, vv, 0)
        pltpu.sync_copy(tchunkf, tbl_hbm.at[pl.ds(f0 * 16, FC * 16)])
        return 0

    lax.fori_loop(0, FPT // FC, tb_chunk, 0)


@functools.partial(
    pl.kernel,
    out_type=jax.ShapeDtypeStruct((PX * 4,), jnp.float32),
    mesh=_mesh,
    compiler_params=_params,
    scratch_types=[
        pltpu.VMEM((PB,), jnp.int32),
        pltpu.VMEM((PB * 3,), jnp.float32),
        pltpu.VMEM((PB,), jnp.int32),
        pltpu.VMEM((PB, 16), jnp.float32),
        pltpu.VMEM((PB * 16,), jnp.float32),
        pltpu.VMEM((PB * 4,), jnp.float32),
        pltpu.SemaphoreType.DMA,
    ],
)
def _render(tbl_hbm, p2ft_hbm, bary_hbm, out_hbm,
            p2fb, baryb, idxb, rows2d, rowsf, outbf, sem):
    wid = lax.axis_index("c") * NS + lax.axis_index("s")
    iota = lax.iota(jnp.int32, L)

    def px_chunk(c, _):
        p0 = wid * PXT + c * PB
        pltpu.sync_copy(p2ft_hbm.at[pl.ds(p0, PB)], p2fb)
        pltpu.sync_copy(bary_hbm.at[pl.ds(p0 * 3, PB * 3)], baryb)

        def ig(g, _):
            v = p2fb[pl.ds(g * L, L)]
            idxb[pl.ds(g * L, L)] = jnp.maximum(v, 0)
            return 0

        lax.fori_loop(0, PB // L, ig, 0)
        pltpu.async_copy(tbl_hbm.at[idxb], rows2d, sem).wait()

        def flat8(k, _):
            for u in range(8):
                p = k * 8 + u
                rowsf[pl.ds(p * L, L)] = rows2d[p, :]
            return 0

        lax.fori_loop(0, PB // 8, flat8, 0)

        def grp(g, _):
            base = g * L
            r = base + iota
            r16 = r * 16

            def colg(cc):
                return plsc.load_gather(rowsf, [r16 + cc])

            b0 = plsc.load_gather(baryb, [r * 3])
            b1 = plsc.load_gather(baryb, [r * 3 + 1])
            b2 = plsc.load_gather(baryb, [r * 3 + 2])
            ox = b0 * colg(0) + b1 * colg(4) + b2 * colg(8)
            oy = b0 * colg(1) + b1 * colg(5) + b2 * colg(9)
            oz = b0 * colg(2) + b1 * colg(6) + b2 * colg(10)
            vcnt = colg(12)
            pm = p2fb[pl.ds(base, L)] >= 0
            q = p0 + r
            gx = (q & 511).astype(jnp.float32) * _GRID_SCALE - 1.0
            gy = ((q >> 9) & 511).astype(jnp.float32) * _GRID_SCALE - 1.0
            zero = jnp.zeros((L,), jnp.float32)
            one = jnp.full((L,), 1.0, jnp.float32)
            ox = jnp.where(pm, ox, zero) + gx
            oy = jnp.where(pm, oy, zero) + gy
            oz = jnp.where(pm, oz, zero)
            ov = jnp.where(pm & (vcnt > 0.5), one, zero)
            r4 = r * 4
            for cc, vec in ((0, ox), (1, oy), (2, oz), (3, ov)):
                plsc.store_scatter(outbf, [r4 + cc], vec)
            return 0

        lax.fori_loop(0, PB // L, grp, 0)
        pltpu.sync_copy(outbf, out_hbm.at[pl.ds(p0 * 4, PB * 4)])
        return 0

    lax.fori_loop(0, PXT // PB, px_chunk, 0)


def kernel(verts_source_ndc, verts_target_ndc, faces, pix_to_face_source,
           pix_to_face_target, bary_coords):
    zpad = jnp.zeros(((VPAD - NV) * 3,), jnp.float32)
    vs = jnp.concatenate([verts_source_ndc.reshape(-1), zpad])
    vt = jnp.concatenate([verts_target_ndc.reshape(-1), zpad])
    faces_flat = jnp.concatenate(
        [faces, jnp.zeros((FPAD - F, 3), jnp.int32)], axis=0).reshape(-1)
    p2fs = pix_to_face_source.reshape(-1)
    p2ft = pix_to_face_target.reshape(-1)
    bary = bary_coords.reshape(-1)
    zeros1 = jnp.zeros((FPAD // NS,), jnp.float32)
    zeros16 = jnp.zeros((RC * 16,), jnp.float32)

    fa, fb, fc = _flow_tables(vs, vt, zeros16)
    tbl = _build_table(
        faces_flat, p2fs,
        fa.reshape(VPAD, 16), fb.reshape(VPAD, 16), fc.reshape(VPAD, 16),
        zeros1)
    out = _render(tbl.reshape(FPAD, 16), p2ft, bary)
    return out.reshape(N, H, W, 4)
